# baseline (speedup 1.0000x reference)
import functools

import jax
import jax.numpy as jnp
from jax import lax
from jax.experimental import pallas as pl
from jax.experimental.pallas import tpu as pltpu
from jax.experimental.pallas import tpu_sc as plsc

NUM_CORES = 2
NUM_SUBCORES = 16
NW = NUM_CORES * NUM_SUBCORES
BATCH = 16384
EMBED = 32
BPW = BATCH // NW
CHUNK = 128
NCHUNK = BPW // CHUNK

_MESH = plsc.VectorSubcoreMesh(core_axis_name="c", subcore_axis_name="s")


def _gather_body(uidx_hbm, midx_hbm, ut_hbm, mt_hbm, ue_hbm, me_hbm,
                 idx_v, urows_v, mrows_v, sem):
    wid = lax.axis_index("s") * NUM_CORES + lax.axis_index("c")
    pltpu.sync_copy(uidx_hbm.at[wid], idx_v.at[0])
    pltpu.sync_copy(midx_hbm.at[wid], idx_v.at[1])
    copies = []
    for j in range(NCHUNK):
        copies.append(pltpu.async_copy(
            ut_hbm.at[idx_v.at[0, j]], urows_v.at[pl.ds(j * CHUNK, CHUNK)], sem))
        copies.append(pltpu.async_copy(
            mt_hbm.at[idx_v.at[1, j]], mrows_v.at[pl.ds(j * CHUNK, CHUNK)], sem))
    for c in copies:
        c.wait()
    base = wid * BPW
    pltpu.sync_copy(urows_v, ue_hbm.at[pl.ds(base, BPW)])
    pltpu.sync_copy(mrows_v, me_hbm.at[pl.ds(base, BPW)])


_gather = functools.partial(
    pl.kernel,
    out_type=(
        jax.ShapeDtypeStruct((BATCH, EMBED), jnp.float32),
        jax.ShapeDtypeStruct((BATCH, EMBED), jnp.float32),
    ),
    mesh=_MESH,
    scratch_types=[
        pltpu.VMEM((2, NCHUNK, CHUNK), jnp.int32),
        pltpu.VMEM((BPW, EMBED), jnp.float32),
        pltpu.VMEM((BPW, EMBED), jnp.float32),
        pltpu.SemaphoreType.DMA,
    ],
    compiler_params=pltpu.CompilerParams(use_tc_tiling_on_sc=False),
)(_gather_body)


BS = 2048


def _mlp_body(ue_ref, me_ref, w1u_ref, w1m_ref, b1_ref, w2_ref, b2_ref, out_ref):
    dn = (((1,), (1,)), ((), ()))
    h = lax.dot_general(ue_ref[...], w1u_ref[...], dn,
                        preferred_element_type=jnp.float32)
    h = h + lax.dot_general(me_ref[...], w1m_ref[...], dn,
                            preferred_element_type=jnp.float32)
    h = jnp.maximum(h + b1_ref[...], 0.0)
    out_ref[...] = jnp.sum(h * w2_ref[...], axis=1, keepdims=True) + b2_ref[...]


def _mlp(ue, me, w1u, w1m, b1r, w2, b2r):
    grid = (BATCH // BS,)
    return pl.pallas_call(
        _mlp_body,
        grid=grid,
        in_specs=[
            pl.BlockSpec((BS, EMBED), lambda i: (i, 0)),
            pl.BlockSpec((BS, EMBED), lambda i: (i, 0)),
            pl.BlockSpec((128, EMBED), lambda i: (0, 0)),
            pl.BlockSpec((128, EMBED), lambda i: (0, 0)),
            pl.BlockSpec((1, 128), lambda i: (0, 0)),
            pl.BlockSpec((1, 128), lambda i: (0, 0)),
            pl.BlockSpec((1, 1), lambda i: (0, 0)),
        ],
        out_specs=pl.BlockSpec((BS, 1), lambda i: (i, 0)),
        out_shape=jax.ShapeDtypeStruct((BATCH, 1), jnp.float32),
    )(ue, me, w1u, w1m, b1r, w2, b2r)


def kernel(user, movie, user_table, movie_table, W1, b1, W2, b2):
    uidx = user.astype(jnp.int32).reshape(NW, NCHUNK, CHUNK)
    midx = movie.astype(jnp.int32).reshape(NW, NCHUNK, CHUNK)
    ue, me = _gather(uidx, midx, user_table, movie_table)
    w1u = W1[:, :EMBED]
    w1m = W1[:, EMBED:]
    b1r = b1.reshape(1, 128)
    b2r = b2.reshape(1, 1)
    return _mlp(ue, me, w1u, w1m, b1r, W2, b2r)


# speedup vs baseline: 2.9641x; 2.9641x over previous
import functools

import jax
import jax.numpy as jnp
from jax import lax
from jax.experimental import pallas as pl
from jax.experimental.pallas import tpu as pltpu
from jax.experimental.pallas import tpu_sc as plsc

NUM_CORES = 2
NUM_SUBCORES = 16
NW = NUM_CORES * NUM_SUBCORES
BATCH = 16384
EMBED = 32
NROWS = 1000000
BPW = BATCH // NW
G = 8
NGROUPS = BPW // 16

_MESH = plsc.VectorSubcoreMesh(core_axis_name="c", subcore_axis_name="s")


def _gather_body(uidx_hbm, midx_hbm, ut_hbm, mt_hbm, ue_hbm, me_hbm,
                 idx_v, sbuf_v, obuf_v, gsem, ssem):
    wid = lax.axis_index("s") * NUM_CORES + lax.axis_index("c")
    pltpu.sync_copy(uidx_hbm.at[wid], idx_v.at[0])
    pltpu.sync_copy(midx_hbm.at[wid], idx_v.at[1])
    base = wid * BPW
    iota = lax.iota(jnp.int32, 16)
    for t, (tab, out) in enumerate(((ut_hbm, ue_hbm), (mt_hbm, me_hbm))):
        def group(g, _):
            idxvec = idx_v[t, pl.ds(g * 16, 16)]
            colv = idxvec & 127
            for h in range(2):
                copies = []
                for k in range(G):
                    blk = pl.multiple_of((idxvec[h * G + k] >> 7) << 7, 128)
                    copies.append(pltpu.make_async_copy(
                        tab.at[:, pl.ds(blk, 128)], sbuf_v.at[k], gsem))
                    copies[-1].start()
                for k in range(G):
                    copies[k].wait()
                    cv = jnp.full((16,), colv[h * G + k], jnp.int32)
                    lo = plsc.load_gather(sbuf_v.at[k], [iota, cv])
                    hi = plsc.load_gather(sbuf_v.at[k], [iota + 16, cv])
                    row = g * 16 + h * G + k
                    obuf_v[row, pl.ds(0, 16)] = lo
                    obuf_v[row, pl.ds(16, 16)] = hi
            return 0
        lax.fori_loop(0, NGROUPS, group, 0)
        pltpu.async_copy(obuf_v, out.at[pl.ds(base, BPW)], ssem).wait()


_gather = functools.partial(
    pl.kernel,
    out_type=(
        jax.ShapeDtypeStruct((BATCH, EMBED), jnp.float32),
        jax.ShapeDtypeStruct((BATCH, EMBED), jnp.float32),
    ),
    mesh=_MESH,
    scratch_types=[
        pltpu.VMEM((2, BPW), jnp.int32),
        pltpu.VMEM((G, EMBED, 128), jnp.float32),
        pltpu.VMEM((BPW, EMBED), jnp.float32),
        pltpu.SemaphoreType.DMA,
        pltpu.SemaphoreType.DMA,
    ],
    compiler_params=pltpu.CompilerParams(use_tc_tiling_on_sc=True,
                                         needs_layout_passes=False),
)(_gather_body)


BS = 2048


def _mlp_body(ue_ref, me_ref, w1u_ref, w1m_ref, b1_ref, w2_ref, b2_ref, out_ref):
    dn = (((1,), (1,)), ((), ()))
    h = lax.dot_general(ue_ref[...], w1u_ref[...], dn,
                        preferred_element_type=jnp.float32)
    h = h + lax.dot_general(me_ref[...], w1m_ref[...], dn,
                            preferred_element_type=jnp.float32)
    h = jnp.maximum(h + b1_ref[...], 0.0)
    out_ref[...] = jnp.sum(h * w2_ref[...], axis=1, keepdims=True) + b2_ref[...]


def _mlp(ue, me, w1u, w1m, b1r, w2, b2r):
    grid = (BATCH // BS,)
    return pl.pallas_call(
        _mlp_body,
        grid=grid,
        in_specs=[
            pl.BlockSpec((BS, EMBED), lambda i: (i, 0)),
            pl.BlockSpec((BS, EMBED), lambda i: (i, 0)),
            pl.BlockSpec((128, EMBED), lambda i: (0, 0)),
            pl.BlockSpec((128, EMBED), lambda i: (0, 0)),
            pl.BlockSpec((1, 128), lambda i: (0, 0)),
            pl.BlockSpec((1, 128), lambda i: (0, 0)),
            pl.BlockSpec((1, 1), lambda i: (0, 0)),
        ],
        out_specs=pl.BlockSpec((BS, 1), lambda i: (i, 0)),
        out_shape=jax.ShapeDtypeStruct((BATCH, 1), jnp.float32),
    )(ue, me, w1u, w1m, b1r, w2, b2r)


def kernel(user, movie, user_table, movie_table, W1, b1, W2, b2):
    user = user.astype(jnp.int32)
    movie = movie.astype(jnp.int32)
    ue, me = _gather(user.reshape(NW, BPW), movie.reshape(NW, BPW),
                     user_table.T, movie_table.T)
    w1u = W1[:, :EMBED]
    w1m = W1[:, EMBED:]
    b1r = b1.reshape(1, 128)
    b2r = b2.reshape(1, 1)
    return _mlp(ue, me, w1u, w1m, b1r, W2, b2r)


# speedup vs baseline: 3.5679x; 1.2037x over previous
import functools

import jax
import jax.numpy as jnp
from jax import lax
from jax.experimental import pallas as pl
from jax.experimental.pallas import tpu as pltpu
from jax.experimental.pallas import tpu_sc as plsc

NUM_CORES = 2
NUM_SUBCORES = 16
NW = NUM_CORES * NUM_SUBCORES
BATCH = 16384
EMBED = 32
BPW = BATCH // NW
G = 8
NGROUPS = BPW // 16

_MESH = plsc.VectorSubcoreMesh(core_axis_name="c", subcore_axis_name="s")


def _gather_body(uidx_hbm, midx_hbm, ut_hbm, mt_hbm, ue_hbm, me_hbm,
                 idx_v, sbuf_v, ebuf_v, gsem, ssem):
    wid = lax.axis_index("s") * NUM_CORES + lax.axis_index("c")
    pltpu.sync_copy(uidx_hbm.at[wid], idx_v.at[0])
    pltpu.sync_copy(midx_hbm.at[wid], idx_v.at[1])
    base = wid * BPW
    iota = lax.iota(jnp.int32, 16)
    for t, (tab, out) in enumerate(((ut_hbm, ue_hbm), (mt_hbm, me_hbm))):
        def group(g, _):
            idxvec = idx_v[t, pl.ds(g * 16, 16)]
            colv = idxvec & 127
            copies = []
            for h in range(2):
                for k in range(G):
                    blk = pl.multiple_of((idxvec[h * G + k] >> 7) << 7, 128)
                    copies.append(pltpu.make_async_copy(
                        tab.at[:, pl.ds(blk, 128)], sbuf_v.at[h, k], gsem))
                    copies[-1].start()
            for h in range(2):
                @pl.when(g > 0)
                def _():
                    pltpu.make_async_copy(
                        ebuf_v.at[h], out.at[pl.ds(0, G)], ssem).wait()
                for k in range(G):
                    copies[h * G + k].wait()
                    cv = jnp.full((16,), colv[h * G + k], jnp.int32)
                    lo = plsc.load_gather(sbuf_v.at[h, k], [iota, cv])
                    hi = plsc.load_gather(sbuf_v.at[h, k], [iota + 16, cv])
                    ebuf_v[h, k, pl.ds(0, 16)] = lo
                    ebuf_v[h, k, pl.ds(16, 16)] = hi
                pltpu.make_async_copy(
                    ebuf_v.at[h],
                    out.at[pl.ds(base + g * 16 + h * G, G)], ssem).start()
            return 0
        lax.fori_loop(0, NGROUPS, group, 0)
        for _ in range(2):
            pltpu.make_async_copy(
                ebuf_v.at[0], out.at[pl.ds(0, G)], ssem).wait()


_gather = functools.partial(
    pl.kernel,
    out_type=(
        jax.ShapeDtypeStruct((BATCH, EMBED), jnp.float32),
        jax.ShapeDtypeStruct((BATCH, EMBED), jnp.float32),
    ),
    mesh=_MESH,
    scratch_types=[
        pltpu.VMEM((2, BPW), jnp.int32),
        pltpu.VMEM((2, G, EMBED, 128), jnp.float32),
        pltpu.VMEM((2, G, EMBED), jnp.float32),
        pltpu.SemaphoreType.DMA,
        pltpu.SemaphoreType.DMA,
    ],
    compiler_params=pltpu.CompilerParams(use_tc_tiling_on_sc=True,
                                         needs_layout_passes=False),
)(_gather_body)


BS = 2048


def _mlp_body(ue_ref, me_ref, w1u_ref, w1m_ref, b1_ref, w2_ref, b2_ref, out_ref):
    dn = (((1,), (1,)), ((), ()))
    h = lax.dot_general(ue_ref[...], w1u_ref[...], dn,
                        preferred_element_type=jnp.float32)
    h = h + lax.dot_general(me_ref[...], w1m_ref[...], dn,
                            preferred_element_type=jnp.float32)
    h = jnp.maximum(h + b1_ref[...], 0.0)
    out_ref[...] = jnp.sum(h * w2_ref[...], axis=1, keepdims=True) + b2_ref[...]


def _mlp(ue, me, w1u, w1m, b1r, w2, b2r):
    grid = (BATCH // BS,)
    return pl.pallas_call(
        _mlp_body,
        grid=grid,
        in_specs=[
            pl.BlockSpec((BS, EMBED), lambda i: (i, 0)),
            pl.BlockSpec((BS, EMBED), lambda i: (i, 0)),
            pl.BlockSpec((128, EMBED), lambda i: (0, 0)),
            pl.BlockSpec((128, EMBED), lambda i: (0, 0)),
            pl.BlockSpec((1, 128), lambda i: (0, 0)),
            pl.BlockSpec((1, 128), lambda i: (0, 0)),
            pl.BlockSpec((1, 1), lambda i: (0, 0)),
        ],
        out_specs=pl.BlockSpec((BS, 1), lambda i: (i, 0)),
        out_shape=jax.ShapeDtypeStruct((BATCH, 1), jnp.float32),
    )(ue, me, w1u, w1m, b1r, w2, b2r)


def kernel(user, movie, user_table, movie_table, W1, b1, W2, b2):
    user = user.astype(jnp.int32)
    movie = movie.astype(jnp.int32)
    ue, me = _gather(user.reshape(NW, BPW), movie.reshape(NW, BPW),
                     user_table.T, movie_table.T)
    w1u = W1[:, :EMBED]
    w1m = W1[:, EMBED:]
    b1r = b1.reshape(1, 128)
    b2r = b2.reshape(1, 1)
    return _mlp(ue, me, w1u, w1m, b1r, W2, b2r)


# speedup vs baseline: 4.1719x; 1.1693x over previous
import functools

import jax
import jax.numpy as jnp
from jax import lax
from jax.experimental import pallas as pl
from jax.experimental.pallas import tpu as pltpu
from jax.experimental.pallas import tpu_sc as plsc

NUM_CORES = 2
NUM_SUBCORES = 16
NW = NUM_CORES * NUM_SUBCORES
BATCH = 16384
EMBED = 32
BPW = BATCH // NW
G = 8
NGROUPS = BPW // 16

_MESH = plsc.VectorSubcoreMesh(core_axis_name="c", subcore_axis_name="s")


def _gather_body(uidx_hbm, midx_hbm, ut_hbm, mt_hbm, ue_hbm, me_hbm,
                 idx_v, sbuf_v, ebuf_v, gsem, ssem):
    wid = lax.axis_index("s") * NUM_CORES + lax.axis_index("c")
    pltpu.sync_copy(uidx_hbm.at[wid], idx_v.at[0, pl.ds(0, BPW)])
    pltpu.sync_copy(midx_hbm.at[wid], idx_v.at[1, pl.ds(0, BPW)])
    base = wid * BPW
    iota = lax.iota(jnp.int32, 16)

    def issue_half(tab, idxvec, h):
        for k in range(G):
            blk = pl.multiple_of((idxvec[h * G + k] >> 7) << 7, 128)
            pltpu.make_async_copy(
                tab.at[:, pl.ds(blk, 128)], sbuf_v.at[h, k], gsem).start()

    for t, (tab, out) in enumerate(((ut_hbm, ue_hbm), (mt_hbm, me_hbm))):
        idx0 = idx_v[t, pl.ds(0, 16)]
        issue_half(tab, idx0, 0)
        issue_half(tab, idx0, 1)

        def group(g, idxcur):
            colv = idxcur & 127
            idxnext = idx_v[t, pl.ds((g + 1) * 16, 16)]
            for h in range(2):
                @pl.when(g > 0)
                def _():
                    pltpu.make_async_copy(
                        ebuf_v.at[h], out.at[pl.ds(0, G)], ssem).wait()
                for k in range(G):
                    pltpu.make_async_copy(
                        tab.at[:, pl.ds(0, 128)], sbuf_v.at[h, k], gsem).wait()
                    cv = jnp.full((16,), colv[h * G + k], jnp.int32)
                    lo = plsc.load_gather(sbuf_v.at[h, k], [iota, cv])
                    hi = plsc.load_gather(sbuf_v.at[h, k], [iota + 16, cv])
                    ebuf_v[h, k, pl.ds(0, 16)] = lo
                    ebuf_v[h, k, pl.ds(16, 16)] = hi
                pltpu.make_async_copy(
                    ebuf_v.at[h],
                    out.at[pl.ds(base + g * 16 + h * G, G)], ssem).start()

                @pl.when(g + 1 < NGROUPS)
                def _():
                    issue_half(tab, idxnext, h)
            return idxnext

        lax.fori_loop(0, NGROUPS, group, idx0)
        for _ in range(2):
            pltpu.make_async_copy(
                ebuf_v.at[0], out.at[pl.ds(0, G)], ssem).wait()


_gather = functools.partial(
    pl.kernel,
    out_type=(
        jax.ShapeDtypeStruct((BATCH, EMBED), jnp.float32),
        jax.ShapeDtypeStruct((BATCH, EMBED), jnp.float32),
    ),
    mesh=_MESH,
    scratch_types=[
        pltpu.VMEM((2, BPW + 16), jnp.int32),
        pltpu.VMEM((2, G, EMBED, 128), jnp.float32),
        pltpu.VMEM((2, G, EMBED), jnp.float32),
        pltpu.SemaphoreType.DMA,
        pltpu.SemaphoreType.DMA,
    ],
    compiler_params=pltpu.CompilerParams(use_tc_tiling_on_sc=True,
                                         needs_layout_passes=False),
)(_gather_body)


BS = 2048


def _mlp_body(ue_ref, me_ref, w1u_ref, w1m_ref, b1_ref, w2_ref, b2_ref, out_ref):
    dn = (((1,), (1,)), ((), ()))
    h = lax.dot_general(ue_ref[...], w1u_ref[...], dn,
                        preferred_element_type=jnp.float32)
    h = h + lax.dot_general(me_ref[...], w1m_ref[...], dn,
                            preferred_element_type=jnp.float32)
    h = jnp.maximum(h + b1_ref[...], 0.0)
    out_ref[...] = jnp.sum(h * w2_ref[...], axis=1, keepdims=True) + b2_ref[...]


def _mlp(ue, me, w1u, w1m, b1r, w2, b2r):
    grid = (BATCH // BS,)
    return pl.pallas_call(
        _mlp_body,
        grid=grid,
        in_specs=[
            pl.BlockSpec((BS, EMBED), lambda i: (i, 0)),
            pl.BlockSpec((BS, EMBED), lambda i: (i, 0)),
            pl.BlockSpec((128, EMBED), lambda i: (0, 0)),
            pl.BlockSpec((128, EMBED), lambda i: (0, 0)),
            pl.BlockSpec((1, 128), lambda i: (0, 0)),
            pl.BlockSpec((1, 128), lambda i: (0, 0)),
            pl.BlockSpec((1, 1), lambda i: (0, 0)),
        ],
        out_specs=pl.BlockSpec((BS, 1), lambda i: (i, 0)),
        out_shape=jax.ShapeDtypeStruct((BATCH, 1), jnp.float32),
    )(ue, me, w1u, w1m, b1r, w2, b2r)


def kernel(user, movie, user_table, movie_table, W1, b1, W2, b2):
    user = user.astype(jnp.int32)
    movie = movie.astype(jnp.int32)
    ue, me = _gather(user.reshape(NW, BPW), movie.reshape(NW, BPW),
                     user_table.T, movie_table.T)
    w1u = W1[:, :EMBED]
    w1m = W1[:, EMBED:]
    b1r = b1.reshape(1, 128)
    b2r = b2.reshape(1, 1)
    return _mlp(ue, me, w1u, w1m, b1r, W2, b2r)
